# baseline (speedup 1.0000x reference)
import jax
import jax.numpy as jnp
from jax.experimental import pallas as pl
from jax.experimental.pallas import tpu as pltpu

N_TOK = 32768
DIM = 128
NSEG = 16
TILE = 4096
GRID = N_TOK // TILE


def _leaky(x):
    return jnp.maximum(x, 0.2 * x)


def _smem_to_col(ref, n, offset=0):
    sub = jax.lax.broadcasted_iota(jnp.int32, (n, 1), 0)
    col = jnp.zeros((n, 1), ref.dtype)
    for s in range(n):
        col = jnp.where(sub == s, ref[s + offset], col)
    return col


def _fused_kernel(ptr_ref, b1_ref, b2_ref, x_ref, w1_ref, w2_ref,
                  w3_ref, b3_ref, out_ref, acc_ref, lo_ref, hi_ref,
                  b1c_ref, b2c_ref):
    pid = pl.program_id(0)

    @pl.when(pid == 0)
    def _init():
        acc_ref[...] = jnp.zeros_like(acc_ref)
        lo_ref[...] = _smem_to_col(ptr_ref, NSEG)
        hi_ref[...] = _smem_to_col(ptr_ref, NSEG, offset=1)
        b1c_ref[...] = _smem_to_col(b1_ref, 32)
        b2c_ref[...] = _smem_to_col(b2_ref, NSEG)

    h1t = _leaky(
        jax.lax.dot_general(w1_ref[...].astype(jnp.bfloat16),
                            x_ref[...].astype(jnp.bfloat16),
                            (((0,), (1,)), ((), ())),
                            preferred_element_type=jnp.float32)
        + b1c_ref[...])
    h2t = _leaky(
        jax.lax.dot_general(w2_ref[...], h1t,
                            (((0,), (0,)), ((), ())),
                            preferred_element_type=jnp.float32)
        + b2c_ref[...])

    cols = jax.lax.broadcasted_iota(jnp.int32, (NSEG, TILE), 1) + pid * TILE
    m = jnp.logical_and(cols >= lo_ref[...], cols < hi_ref[...])

    acc_ref[...] += jax.lax.dot_general(
        m.astype(jnp.bfloat16), h2t.astype(jnp.bfloat16),
        (((1,), (1,)), ((), ())),
        preferred_element_type=jnp.float32)

    @pl.when(pid == GRID - 1)
    def _finish():
        cnt = (hi_ref[...] - lo_ref[...]).astype(jnp.float32)
        out_ref[...] = (
            jnp.dot(acc_ref[...], w3_ref[...], preferred_element_type=jnp.float32)
            + cnt * b3_ref[...]
        )


def kernel(h_dag, obs_ptr, W1, b1, W2, b2, W3, b3):
    const = lambda i, *refs: (0, 0)
    grid_spec = pltpu.PrefetchScalarGridSpec(
        num_scalar_prefetch=3,
        grid=(GRID,),
        in_specs=[
            pl.BlockSpec((TILE, DIM), lambda i, *refs: (i, 0)),
            pl.BlockSpec((DIM, 32), const),
            pl.BlockSpec((32, 16), const),
            pl.BlockSpec((16, DIM), const),
            pl.BlockSpec((1, DIM), const),
        ],
        out_specs=pl.BlockSpec((NSEG, DIM), const),
        scratch_shapes=[
            pltpu.VMEM((NSEG, 16), jnp.float32),
            pltpu.VMEM((NSEG, 1), jnp.int32),
            pltpu.VMEM((NSEG, 1), jnp.int32),
            pltpu.VMEM((32, 1), jnp.float32),
            pltpu.VMEM((NSEG, 1), jnp.float32),
        ],
    )
    out = pl.pallas_call(
        _fused_kernel,
        grid_spec=grid_spec,
        out_shape=jax.ShapeDtypeStruct((NSEG, DIM), jnp.float32),
        compiler_params=pltpu.CompilerParams(
            dimension_semantics=("arbitrary",),
        ),
    )(obs_ptr, b1, b2, h_dag, W1, W2, W3, b3.reshape(1, DIM))
    return out


# speedup vs baseline: 1.2132x; 1.2132x over previous
import jax
import jax.numpy as jnp
from jax.experimental import pallas as pl
from jax.experimental.pallas import tpu as pltpu

N_TOK = 32768
DIM = 128
NSEG = 16
TILE = 16384
GRID = N_TOK // TILE


def _leaky(x):
    return jnp.maximum(x, 0.2 * x)


def _smem_to_col(ref, n, offset=0):
    sub = jax.lax.broadcasted_iota(jnp.int32, (n, 1), 0)
    col = jnp.zeros((n, 1), ref.dtype)
    for s in range(n):
        col = jnp.where(sub == s, ref[s + offset], col)
    return col


def _fused_kernel(ptr_ref, b1_ref, b2_ref, x_ref, w1_ref, w2_ref,
                  w3_ref, b3_ref, out_ref, acc_ref, lo_ref, hi_ref,
                  b1c_ref, b2c_ref):
    pid = pl.program_id(0)

    @pl.when(pid == 0)
    def _init():
        acc_ref[...] = jnp.zeros_like(acc_ref)
        lo_ref[...] = _smem_to_col(ptr_ref, NSEG)
        hi_ref[...] = _smem_to_col(ptr_ref, NSEG, offset=1)
        b1c_ref[...] = _smem_to_col(b1_ref, 32)
        b2c_ref[...] = _smem_to_col(b2_ref, NSEG)

    h1t = _leaky(
        jax.lax.dot_general(w1_ref[...].astype(jnp.bfloat16),
                            x_ref[...].astype(jnp.bfloat16),
                            (((0,), (1,)), ((), ())),
                            preferred_element_type=jnp.float32)
        + b1c_ref[...])
    h2t = _leaky(
        jax.lax.dot_general(w2_ref[...], h1t,
                            (((0,), (0,)), ((), ())),
                            preferred_element_type=jnp.float32)
        + b2c_ref[...])

    cols = jax.lax.broadcasted_iota(jnp.int32, (NSEG, TILE), 1) + pid * TILE
    m = jnp.logical_and(cols >= lo_ref[...], cols < hi_ref[...])

    acc_ref[...] += jax.lax.dot_general(
        m.astype(jnp.bfloat16), h2t.astype(jnp.bfloat16),
        (((1,), (1,)), ((), ())),
        preferred_element_type=jnp.float32)

    @pl.when(pid == GRID - 1)
    def _finish():
        cnt = (hi_ref[...] - lo_ref[...]).astype(jnp.float32)
        out_ref[...] = (
            jnp.dot(acc_ref[...], w3_ref[...], preferred_element_type=jnp.float32)
            + cnt * b3_ref[...]
        )


def kernel(h_dag, obs_ptr, W1, b1, W2, b2, W3, b3):
    const = lambda i, *refs: (0, 0)
    grid_spec = pltpu.PrefetchScalarGridSpec(
        num_scalar_prefetch=3,
        grid=(GRID,),
        in_specs=[
            pl.BlockSpec((TILE, DIM), lambda i, *refs: (i, 0)),
            pl.BlockSpec((DIM, 32), const),
            pl.BlockSpec((32, 16), const),
            pl.BlockSpec((16, DIM), const),
            pl.BlockSpec((1, DIM), const),
        ],
        out_specs=pl.BlockSpec((NSEG, DIM), const),
        scratch_shapes=[
            pltpu.VMEM((NSEG, 16), jnp.float32),
            pltpu.VMEM((NSEG, 1), jnp.int32),
            pltpu.VMEM((NSEG, 1), jnp.int32),
            pltpu.VMEM((32, 1), jnp.float32),
            pltpu.VMEM((NSEG, 1), jnp.float32),
        ],
    )
    out = pl.pallas_call(
        _fused_kernel,
        grid_spec=grid_spec,
        out_shape=jax.ShapeDtypeStruct((NSEG, DIM), jnp.float32),
        compiler_params=pltpu.CompilerParams(
            dimension_semantics=("arbitrary",),
        ),
    )(obs_ptr, b1, b2, h_dag, W1, W2, W3, b3.reshape(1, DIM))
    return out
